# Initial kernel scaffold; baseline (speedup 1.0000x reference)
import jax
import jax.numpy as jnp
from jax.experimental import pallas as pl


def kernel(x, codebooks):
    raise NotImplementedError("write your pallas kernel here")


# speedup vs baseline: 1.4362x; 1.4362x over previous
import functools

import jax
import jax.numpy as jnp
from jax import lax
from jax.experimental import pallas as pl
from jax.experimental.pallas import tpu as pltpu
from jax.experimental.pallas import tpu_sc as plsc

QD = 4
K = 8192
D = 256
N = 16384
TB = 256
NTB = N // TB

NW = 32
BPW = N // NW
CH = 128
NCH = BPW // CH


def _vq_step_kernel(r_ref, q_ref, cb_ref, cbn_ref, idx_ref, rout_ref):
    r = r_ref[...] - q_ref[...]
    cb = cb_ref[...]
    rnorm = jnp.sum(r * r, axis=1, keepdims=True)
    s = jax.lax.dot_general(r.astype(jnp.bfloat16), cb.astype(jnp.bfloat16),
                            (((1,), (1,)), ((), ())),
                            preferred_element_type=jnp.float32)
    d = (rnorm + cbn_ref[...]) - 2.0 * s
    idx_ref[0, 0, :] = jnp.argmin(d, axis=1)
    rout_ref[...] = r


def _vq_step(r, q_prev, cb, cbn):
    return pl.pallas_call(
        _vq_step_kernel,
        grid=(NTB,),
        in_specs=[
            pl.BlockSpec((TB, D), lambda i: (i, 0)),
            pl.BlockSpec((TB, D), lambda i: (i, 0)),
            pl.BlockSpec((K, D), lambda i: (0, 0)),
            pl.BlockSpec((1, K), lambda i: (0, 0)),
        ],
        out_specs=[
            pl.BlockSpec((1, 1, TB), lambda i: (i, 0, 0)),
            pl.BlockSpec((TB, D), lambda i: (i, 0)),
        ],
        out_shape=[
            jax.ShapeDtypeStruct((NTB, 1, TB), jnp.int32),
            jax.ShapeDtypeStruct((N, D), jnp.float32),
        ],
    )(r, q_prev, cb, cbn)


def _sc_gather(cb, idx):
    mesh = plsc.VectorSubcoreMesh(core_axis_name="c", subcore_axis_name="s")

    @functools.partial(
        pl.kernel,
        mesh=mesh,
        out_type=jax.ShapeDtypeStruct((N, D), jnp.float32),
        scratch_types=[
            pltpu.VMEM((CH,), jnp.int32),
            pltpu.VMEM((CH, D), jnp.float32),
            pltpu.SemaphoreType.DMA,
        ],
    )
    def k(cb_hbm, idx_hbm, out_hbm, idx_v, rows_v, sem):
        wid = lax.axis_index("s") * 2 + lax.axis_index("c")
        base = wid * BPW
        for c in range(NCH):
            off = base + c * CH
            pltpu.sync_copy(idx_hbm.at[pl.ds(off, CH)], idx_v)
            pltpu.async_copy(cb_hbm.at[idx_v], rows_v, sem).wait()
            pltpu.sync_copy(rows_v, out_hbm.at[pl.ds(off, CH)])

    return k(cb, idx)


def _epilogue_kernel(x_ref, q0_ref, q1_ref, q2_ref, q3_ref,
                     qt_ref, loss_ref):
    i = pl.program_id(0)
    x = x_ref[...]
    agg1 = q0_ref[...]
    agg2 = agg1 + q1_ref[...]
    agg3 = agg2 + q2_ref[...]
    agg4 = agg3 + q3_ref[...]
    qt_ref[...] = x + (agg4 - x)
    parts = []
    for agg in (agg1, agg2, agg3, agg4):
        e = (x - agg) ** 2.0
        parts.append(jnp.sum(e.reshape(TB, 2, 128), axis=(0, 1)))
    psum = jnp.stack(parts)

    @pl.when(i == 0)
    def _():
        loss_ref[...] = psum

    @pl.when(i > 0)
    def _():
        loss_ref[...] += psum


def _epilogue(x_flat, qs):
    return pl.pallas_call(
        _epilogue_kernel,
        grid=(NTB,),
        in_specs=[pl.BlockSpec((TB, D), lambda i: (i, 0))] * 5,
        out_specs=[
            pl.BlockSpec((TB, D), lambda i: (i, 0)),
            pl.BlockSpec((QD, 128), lambda i: (0, 0)),
        ],
        out_shape=[
            jax.ShapeDtypeStruct((N, D), jnp.float32),
            jax.ShapeDtypeStruct((QD, 128), jnp.float32),
        ],
    )(x_flat, *qs)


def kernel(x, codebooks):
    x_flat = x.reshape(N, D)
    r = x_flat
    q_prev = jnp.zeros((N, D), jnp.float32)
    idxs = []
    qs = []
    for i in range(QD):
        cb = codebooks[i]
        cbn = jnp.sum(cb ** 2.0, axis=1)[None, :]
        idx3, r = _vq_step(r, q_prev, cb, cbn)
        idx = idx3.reshape(N)
        q_prev = _sc_gather(cb, idx)
        idxs.append(idx)
        qs.append(q_prev)
    qt_flat, loss128 = _epilogue(x_flat, qs)
    losses = jnp.sum(loss128, axis=1) / (N * D)
    commitment_loss = jnp.mean(losses)
    quants_trunc = qt_flat.reshape(x.shape)
    codes = jnp.stack(idxs, axis=-1).reshape(x.shape[:-1] + (QD,))
    return quants_trunc, commitment_loss, codes


# speedup vs baseline: 1.6272x; 1.1330x over previous
import functools

import jax
import jax.numpy as jnp
from jax import lax
from jax.experimental import pallas as pl
from jax.experimental.pallas import tpu as pltpu
from jax.experimental.pallas import tpu_sc as plsc

QD = 4
K = 8192
D = 256
N = 16384
TB = 512
NTB = N // TB

NW = 32
BPW = N // NW
CH = 128
NCH = BPW // CH


def _vq_step_kernel(r_ref, q_ref, cb_ref, cbn_ref, idx_ref, rout_ref):
    r = r_ref[...] - q_ref[...]
    cb = cb_ref[...]
    rnorm = jnp.sum(r * r, axis=1, keepdims=True)
    s = jax.lax.dot_general(r.astype(jnp.bfloat16), cb.astype(jnp.bfloat16),
                            (((1,), (1,)), ((), ())),
                            preferred_element_type=jnp.float32)
    d = (rnorm + cbn_ref[...]) - 2.0 * s
    idx_ref[0, 0, :] = jnp.argmin(d, axis=1)
    rout_ref[...] = r


def _vq_step(r, q_prev, cb, cbn):
    return pl.pallas_call(
        _vq_step_kernel,
        grid=(NTB,),
        in_specs=[
            pl.BlockSpec((TB, D), lambda i: (i, 0)),
            pl.BlockSpec((TB, D), lambda i: (i, 0)),
            pl.BlockSpec((K, D), lambda i: (0, 0)),
            pl.BlockSpec((1, K), lambda i: (0, 0)),
        ],
        out_specs=[
            pl.BlockSpec((1, 1, TB), lambda i: (i, 0, 0)),
            pl.BlockSpec((TB, D), lambda i: (i, 0)),
        ],
        out_shape=[
            jax.ShapeDtypeStruct((NTB, 1, TB), jnp.int32),
            jax.ShapeDtypeStruct((N, D), jnp.float32),
        ],
    )(r, q_prev, cb, cbn)


def _sc_gather(cb, idx):
    mesh = plsc.VectorSubcoreMesh(core_axis_name="c", subcore_axis_name="s")

    @functools.partial(
        pl.kernel,
        mesh=mesh,
        out_type=jax.ShapeDtypeStruct((N, D), jnp.float32),
        scratch_types=[
            pltpu.VMEM((CH,), jnp.int32),
            pltpu.VMEM((CH, D), jnp.float32),
            pltpu.SemaphoreType.DMA,
        ],
    )
    def k(cb_hbm, idx_hbm, out_hbm, idx_v, rows_v, sem):
        wid = lax.axis_index("s") * 2 + lax.axis_index("c")
        base = wid * BPW
        for c in range(NCH):
            off = base + c * CH
            pltpu.sync_copy(idx_hbm.at[pl.ds(off, CH)], idx_v)
            pltpu.async_copy(cb_hbm.at[idx_v], rows_v, sem).wait()
            pltpu.sync_copy(rows_v, out_hbm.at[pl.ds(off, CH)])

    return k(cb, idx)


def _epilogue_kernel(x_ref, q0_ref, q1_ref, q2_ref, q3_ref,
                     qt_ref, loss_ref):
    i = pl.program_id(0)
    x = x_ref[...]
    agg1 = q0_ref[...]
    agg2 = agg1 + q1_ref[...]
    agg3 = agg2 + q2_ref[...]
    agg4 = agg3 + q3_ref[...]
    qt_ref[...] = x + (agg4 - x)
    parts = []
    for agg in (agg1, agg2, agg3, agg4):
        e = (x - agg) ** 2.0
        parts.append(jnp.sum(e.reshape(TB, 2, 128), axis=(0, 1)))
    psum = jnp.stack(parts)

    @pl.when(i == 0)
    def _():
        loss_ref[...] = psum

    @pl.when(i > 0)
    def _():
        loss_ref[...] += psum


def _epilogue(x_flat, qs):
    return pl.pallas_call(
        _epilogue_kernel,
        grid=(NTB,),
        in_specs=[pl.BlockSpec((TB, D), lambda i: (i, 0))] * 5,
        out_specs=[
            pl.BlockSpec((TB, D), lambda i: (i, 0)),
            pl.BlockSpec((QD, 128), lambda i: (0, 0)),
        ],
        out_shape=[
            jax.ShapeDtypeStruct((N, D), jnp.float32),
            jax.ShapeDtypeStruct((QD, 128), jnp.float32),
        ],
    )(x_flat, *qs)


def kernel(x, codebooks):
    x_flat = x.reshape(N, D)
    r = x_flat
    q_prev = jnp.zeros((N, D), jnp.float32)
    idxs = []
    qs = []
    for i in range(QD):
        cb = codebooks[i]
        cbn = jnp.sum(cb ** 2.0, axis=1)[None, :]
        idx3, r = _vq_step(r, q_prev, cb, cbn)
        idx = idx3.reshape(N)
        q_prev = _sc_gather(cb, idx)
        idxs.append(idx)
        qs.append(q_prev)
    qt_flat, loss128 = _epilogue(x_flat, qs)
    losses = jnp.sum(loss128, axis=1) / (N * D)
    commitment_loss = jnp.mean(losses)
    quants_trunc = qt_flat.reshape(x.shape)
    codes = jnp.stack(idxs, axis=-1).reshape(x.shape[:-1] + (QD,))
    return quants_trunc, commitment_loss, codes


# speedup vs baseline: 1.6513x; 1.0148x over previous
import functools

import jax
import jax.numpy as jnp
from jax import lax
from jax.experimental import pallas as pl
from jax.experimental.pallas import tpu as pltpu
from jax.experimental.pallas import tpu_sc as plsc

QD = 4
K = 8192
D = 256
N = 16384
TB = 1024
NTB = N // TB

NW = 32
BPW = N // NW
CH = 128
NCH = BPW // CH


def _vq_step_kernel(r_ref, q_ref, cb_ref, cbn_ref, idx_ref, rout_ref):
    r = r_ref[...] - q_ref[...]
    cb = cb_ref[...]
    rnorm = jnp.sum(r * r, axis=1, keepdims=True)
    s = jax.lax.dot_general(r.astype(jnp.bfloat16), cb.astype(jnp.bfloat16),
                            (((1,), (1,)), ((), ())),
                            preferred_element_type=jnp.float32)
    d = (rnorm + cbn_ref[...]) - 2.0 * s
    idx_ref[0, 0, :] = jnp.argmin(d, axis=1)
    rout_ref[...] = r


def _vq_step(r, q_prev, cb, cbn):
    return pl.pallas_call(
        _vq_step_kernel,
        grid=(NTB,),
        in_specs=[
            pl.BlockSpec((TB, D), lambda i: (i, 0)),
            pl.BlockSpec((TB, D), lambda i: (i, 0)),
            pl.BlockSpec((K, D), lambda i: (0, 0)),
            pl.BlockSpec((1, K), lambda i: (0, 0)),
        ],
        out_specs=[
            pl.BlockSpec((1, 1, TB), lambda i: (i, 0, 0)),
            pl.BlockSpec((TB, D), lambda i: (i, 0)),
        ],
        out_shape=[
            jax.ShapeDtypeStruct((NTB, 1, TB), jnp.int32),
            jax.ShapeDtypeStruct((N, D), jnp.float32),
        ],
    )(r, q_prev, cb, cbn)


def _sc_gather(cb, idx):
    mesh = plsc.VectorSubcoreMesh(core_axis_name="c", subcore_axis_name="s")

    @functools.partial(
        pl.kernel,
        mesh=mesh,
        out_type=jax.ShapeDtypeStruct((N, D), jnp.float32),
        scratch_types=[
            pltpu.VMEM((CH,), jnp.int32),
            pltpu.VMEM((CH, D), jnp.float32),
            pltpu.SemaphoreType.DMA,
        ],
    )
    def k(cb_hbm, idx_hbm, out_hbm, idx_v, rows_v, sem):
        wid = lax.axis_index("s") * 2 + lax.axis_index("c")
        base = wid * BPW
        for c in range(NCH):
            off = base + c * CH
            pltpu.sync_copy(idx_hbm.at[pl.ds(off, CH)], idx_v)
            pltpu.async_copy(cb_hbm.at[idx_v], rows_v, sem).wait()
            pltpu.sync_copy(rows_v, out_hbm.at[pl.ds(off, CH)])

    return k(cb, idx)


def _epilogue_kernel(x_ref, q0_ref, q1_ref, q2_ref, q3_ref,
                     qt_ref, loss_ref):
    i = pl.program_id(0)
    x = x_ref[...]
    agg1 = q0_ref[...]
    agg2 = agg1 + q1_ref[...]
    agg3 = agg2 + q2_ref[...]
    agg4 = agg3 + q3_ref[...]
    qt_ref[...] = x + (agg4 - x)
    parts = []
    for agg in (agg1, agg2, agg3, agg4):
        e = (x - agg) ** 2.0
        parts.append(jnp.sum(e.reshape(TB, 2, 128), axis=(0, 1)))
    psum = jnp.stack(parts)

    @pl.when(i == 0)
    def _():
        loss_ref[...] = psum

    @pl.when(i > 0)
    def _():
        loss_ref[...] += psum


def _epilogue(x_flat, qs):
    return pl.pallas_call(
        _epilogue_kernel,
        grid=(NTB,),
        in_specs=[pl.BlockSpec((TB, D), lambda i: (i, 0))] * 5,
        out_specs=[
            pl.BlockSpec((TB, D), lambda i: (i, 0)),
            pl.BlockSpec((QD, 128), lambda i: (0, 0)),
        ],
        out_shape=[
            jax.ShapeDtypeStruct((N, D), jnp.float32),
            jax.ShapeDtypeStruct((QD, 128), jnp.float32),
        ],
    )(x_flat, *qs)


def kernel(x, codebooks):
    x_flat = x.reshape(N, D)
    r = x_flat
    q_prev = jnp.zeros((N, D), jnp.float32)
    idxs = []
    qs = []
    for i in range(QD):
        cb = codebooks[i]
        cbn = jnp.sum(cb ** 2.0, axis=1)[None, :]
        idx3, r = _vq_step(r, q_prev, cb, cbn)
        idx = idx3.reshape(N)
        q_prev = _sc_gather(cb, idx)
        idxs.append(idx)
        qs.append(q_prev)
    qt_flat, loss128 = _epilogue(x_flat, qs)
    losses = jnp.sum(loss128, axis=1) / (N * D)
    commitment_loss = jnp.mean(losses)
    quants_trunc = qt_flat.reshape(x.shape)
    codes = jnp.stack(idxs, axis=-1).reshape(x.shape[:-1] + (QD,))
    return quants_trunc, commitment_loss, codes
